# Initial kernel scaffold; baseline (speedup 1.0000x reference)
import jax
import jax.numpy as jnp
from jax.experimental import pallas as pl


def kernel(ent_embed, rel_embed, norm, edge_index, ent_id, rel_id, W_r0, W_sl0, W_el0, W_r1, W_sl1, W_el1):
    raise NotImplementedError("write your pallas kernel here")


# speedup vs baseline: 3.8000x; 3.8000x over previous
import functools

import jax
import jax.numpy as jnp
from jax import lax
from jax.experimental import pallas as pl
from jax.experimental.pallas import tpu as pltpu
from jax.experimental.pallas import tpu_sc as plsc

N = 10000
E = 160000
D = 256
R = 200
H = 128
NS = 16
NCHUNK = 80
CHUNK = 125
NP_ = 10240
RPT = NP_ // NS
ZROWS = 128

_MESH = plsc.VectorSubcoreMesh(core_axis_name="c", subcore_axis_name="s")


def _sc_layer(h0, h1, r0, r1, src3, dst3, rel3):
    @functools.partial(
        pl.kernel,
        out_type=jax.ShapeDtypeStruct((2 * NP_, H), jnp.float32),
        mesh=_MESH,
        scratch_types=[
            pltpu.VMEM((NCHUNK // 2, CHUNK), jnp.int32),
            pltpu.VMEM((NCHUNK // 2, CHUNK), jnp.int32),
            pltpu.VMEM((NCHUNK // 2, CHUNK), jnp.int32),
            pltpu.VMEM((ZROWS, H), jnp.float32),
            pltpu.VMEM_SHARED((NP_, H), jnp.float32),
            pltpu.SemaphoreType.DMA,
        ],
    )
    def k(h0_hbm, h1_hbm, r0_hbm, r1_hbm, src_hbm, dst_hbm, rel_hbm, out_hbm,
          src_v, dst_v, rel_v, rows_v, agg_sh, sem):
        c = lax.axis_index("c")
        s = lax.axis_index("s")

        zv = jnp.zeros((16,), jnp.float32)

        def zrow(i, carry):
            rows_v[i // 8, pl.ds((i % 8) * 16, 16)] = zv
            return carry

        lax.fori_loop(0, ZROWS * 8, zrow, 0)
        row0 = s * RPT
        for t in range(5):
            pltpu.sync_copy(rows_v, agg_sh.at[pl.ds(row0 + t * ZROWS, ZROWS)])
        plsc.subcore_barrier()

        def phase(tab_hbm, idx_v):
            def step(j, carry):
                pltpu.async_copy(tab_hbm.at[idx_v.at[j]],
                                 rows_v.at[pl.ds(0, CHUNK)], sem).wait()
                pltpu.sync_copy(rows_v.at[pl.ds(0, CHUNK)],
                                agg_sh.at[dst_v.at[j]], add=True)
                return carry
            lax.fori_loop(0, NCHUNK // 2, step, 0)

        HC = NCHUNK // 2
        for ho in range(2):
            pltpu.sync_copy(src_hbm.at[s].at[pl.ds(ho * HC, HC)], src_v)
            pltpu.sync_copy(dst_hbm.at[s].at[pl.ds(ho * HC, HC)], dst_v)
            pltpu.sync_copy(rel_hbm.at[s].at[pl.ds(ho * HC, HC)], rel_v)

            @pl.when(c == 0)
            def _():
                phase(h0_hbm, src_v)
                phase(r0_hbm, rel_v)

            @pl.when(c == 1)
            def _():
                phase(h1_hbm, src_v)
                phase(r1_hbm, rel_v)

        plsc.subcore_barrier()

        pltpu.sync_copy(agg_sh.at[pl.ds(row0, RPT)],
                        out_hbm.at[pl.ds(c * NP_ + row0, RPT)])

    return k(h0, h1, r0, r1, src3, dst3, rel3)


_BM = 1000


def _tc_transform(S2, W, normv, split_out):
    grid = (N // _BM,)
    in_specs = [
        pl.BlockSpec((2, _BM, H), lambda i: (0, i, 0)),
        pl.BlockSpec((D, D), lambda i: (0, 0)),
        pl.BlockSpec((_BM, 1), lambda i: (i, 0)),
    ]
    if split_out:
        out_shape = (jax.ShapeDtypeStruct((N, H), jnp.float32),
                     jax.ShapeDtypeStruct((N, H), jnp.float32))
        out_specs = (pl.BlockSpec((_BM, H), lambda i: (i, 0)),
                     pl.BlockSpec((_BM, H), lambda i: (i, 0)))
    else:
        out_shape = jax.ShapeDtypeStruct((N, D), jnp.float32)
        out_specs = pl.BlockSpec((_BM, D), lambda i: (i, 0))

    def body(s_ref, w_ref, n_ref, *o_refs):
        s0 = s_ref[0]
        s1 = s_ref[1]
        w = w_ref[...]
        hA = lax.dot_general(s0, w[:, :H], (((1,), (1,)), ((), ())),
                             preferred_element_type=jnp.float32,
                             precision=lax.Precision.HIGHEST)
        hB = lax.dot_general(s1, w[:, H:], (((1,), (1,)), ((), ())),
                             preferred_element_type=jnp.float32,
                             precision=lax.Precision.HIGHEST)
        h = (hA + hB) * n_ref[...]
        if split_out:
            o_refs[0][...] = h[:, :H]
            o_refs[1][...] = h[:, H:]
        else:
            o_refs[0][...] = h

    return pl.pallas_call(body, grid=grid, in_specs=in_specs,
                          out_specs=out_specs, out_shape=out_shape)(S2, W, normv)


def kernel(ent_embed, rel_embed, norm, edge_index, ent_id, rel_id,
           W_r0, W_sl0, W_el0, W_r1, W_sl1, W_el1):
    src3 = edge_index[0].reshape(NS, NCHUNK, CHUNK)
    dst3 = edge_index[1].reshape(NS, NCHUNK, CHUNK)
    rel3 = rel_id.reshape(NS, NCHUNK, CHUNK)
    h0 = ent_embed[:, :H]
    h1 = ent_embed[:, H:]
    r0 = rel_embed[:, :H]
    r1 = rel_embed[:, H:]
    normv = norm

    S0 = _sc_layer(h0, h1, r0, r1, src3, dst3, rel3).reshape(2, NP_, H)
    g0, g1 = _tc_transform(S0, W_r0, normv, split_out=True)
    S1 = _sc_layer(g0, g1, r0, r1, src3, dst3, rel3).reshape(2, NP_, H)
    return _tc_transform(S1, W_r1, normv, split_out=False)


# speedup vs baseline: 4.3977x; 1.1573x over previous
import functools

import jax
import jax.numpy as jnp
from jax import lax
from jax.experimental import pallas as pl
from jax.experimental.pallas import tpu as pltpu
from jax.experimental.pallas import tpu_sc as plsc

N = 10000
E = 160000
D = 256
R = 200
H = 128
NS = 16
NCHUNK = 80
CHUNK = 125
QCH = 16
NP_ = 10240
RPT = NP_ // NS
ZROWS = 128

_MESH = plsc.VectorSubcoreMesh(core_axis_name="c", subcore_axis_name="s")


def _sc_layer(h0, h1, r0, r1, src3, dst3, rel3):
    @functools.partial(
        pl.kernel,
        out_type=jax.ShapeDtypeStruct((2 * NP_, H), jnp.float32),
        mesh=_MESH,
        scratch_types=[
            pltpu.VMEM((QCH, CHUNK), jnp.int32),
            pltpu.VMEM((QCH, CHUNK), jnp.int32),
            pltpu.VMEM((ZROWS, H), jnp.float32),
            pltpu.VMEM((CHUNK, H), jnp.float32),
            pltpu.VMEM_SHARED((NP_, H), jnp.float32),
            pltpu.SemaphoreType.DMA,
            pltpu.SemaphoreType.DMA,
            pltpu.SemaphoreType.DMA,
            pltpu.SemaphoreType.DMA,
        ],
    )
    def k(h0_hbm, h1_hbm, r0_hbm, r1_hbm, src_hbm, dst_hbm, rel_hbm, out_hbm,
          idx_q, dst_q, rows0, rows1, agg_sh, gs0, gs1, ss0, ss1):
        c = lax.axis_index("c")
        s = lax.axis_index("s")

        zv = jnp.zeros((16,), jnp.float32)

        def zrow(i, carry):
            rows0[i // 8, pl.ds((i % 8) * 16, 16)] = zv
            return carry

        lax.fori_loop(0, ZROWS * 8, zrow, 0)
        row0 = s * RPT
        for t in range(5):
            pltpu.sync_copy(rows0, agg_sh.at[pl.ds(row0 + t * ZROWS, ZROWS)])
        plsc.subcore_barrier()

        r0v = rows0.at[pl.ds(0, CHUNK)]

        def phase(tab_hbm, src_idx_hbm):
            for q in range(NCHUNK // QCH):
                pltpu.sync_copy(src_idx_hbm.at[s].at[pl.ds(q * QCH, QCH)], idx_q)
                pltpu.sync_copy(dst_hbm.at[s].at[pl.ds(q * QCH, QCH)], dst_q)
                pltpu.async_copy(tab_hbm.at[idx_q.at[0]], r0v, gs0)
                pltpu.async_copy(tab_hbm.at[idx_q.at[1]], rows1, gs1)

                def body(t, carry):
                    j = 2 * t
                    pltpu.make_async_copy(tab_hbm.at[idx_q.at[j]], r0v, gs0).wait()
                    pltpu.async_copy(r0v, agg_sh.at[dst_q.at[j]], ss0, add=True)
                    pltpu.make_async_copy(tab_hbm.at[idx_q.at[j + 1]],
                                          rows1, gs1).wait()
                    pltpu.async_copy(rows1, agg_sh.at[dst_q.at[j + 1]],
                                     ss1, add=True)
                    pltpu.make_async_copy(r0v, agg_sh.at[dst_q.at[j]], ss0).wait()

                    @pl.when(j + 2 < QCH)
                    def _():
                        pltpu.async_copy(tab_hbm.at[idx_q.at[j + 2]], r0v, gs0)

                    pltpu.make_async_copy(rows1, agg_sh.at[dst_q.at[j + 1]],
                                          ss1).wait()

                    @pl.when(j + 3 < QCH)
                    def _():
                        pltpu.async_copy(tab_hbm.at[idx_q.at[j + 3]], rows1, gs1)
                    return carry

                lax.fori_loop(0, QCH // 2, body, 0)

        @pl.when(c == 0)
        def _():
            phase(h0_hbm, src_hbm)
            phase(r0_hbm, rel_hbm)

        @pl.when(c == 1)
        def _():
            phase(h1_hbm, src_hbm)
            phase(r1_hbm, rel_hbm)

        plsc.subcore_barrier()

        pltpu.sync_copy(agg_sh.at[pl.ds(row0, RPT)],
                        out_hbm.at[pl.ds(c * NP_ + row0, RPT)])

    return k(h0, h1, r0, r1, src3, dst3, rel3)


_BM = 1000


def _tc_transform(S2, W, normv, split_out):
    grid = (N // _BM,)
    in_specs = [
        pl.BlockSpec((2, _BM, H), lambda i: (0, i, 0)),
        pl.BlockSpec((D, D), lambda i: (0, 0)),
        pl.BlockSpec((_BM, 1), lambda i: (i, 0)),
    ]
    if split_out:
        out_shape = (jax.ShapeDtypeStruct((N, H), jnp.float32),
                     jax.ShapeDtypeStruct((N, H), jnp.float32))
        out_specs = (pl.BlockSpec((_BM, H), lambda i: (i, 0)),
                     pl.BlockSpec((_BM, H), lambda i: (i, 0)))
    else:
        out_shape = jax.ShapeDtypeStruct((N, D), jnp.float32)
        out_specs = pl.BlockSpec((_BM, D), lambda i: (i, 0))

    def body(s_ref, w_ref, n_ref, *o_refs):
        s0 = s_ref[0]
        s1 = s_ref[1]
        w = w_ref[...]
        hA = lax.dot_general(s0, w[:, :H], (((1,), (1,)), ((), ())),
                             preferred_element_type=jnp.float32,
                             precision=lax.Precision.HIGHEST)
        hB = lax.dot_general(s1, w[:, H:], (((1,), (1,)), ((), ())),
                             preferred_element_type=jnp.float32,
                             precision=lax.Precision.HIGHEST)
        h = (hA + hB) * n_ref[...]
        if split_out:
            o_refs[0][...] = h[:, :H]
            o_refs[1][...] = h[:, H:]
        else:
            o_refs[0][...] = h

    return pl.pallas_call(body, grid=grid, in_specs=in_specs,
                          out_specs=out_specs, out_shape=out_shape)(S2, W, normv)


def kernel(ent_embed, rel_embed, norm, edge_index, ent_id, rel_id,
           W_r0, W_sl0, W_el0, W_r1, W_sl1, W_el1):
    src3 = edge_index[0].reshape(NS, NCHUNK, CHUNK)
    dst3 = edge_index[1].reshape(NS, NCHUNK, CHUNK)
    rel3 = rel_id.reshape(NS, NCHUNK, CHUNK)
    h0 = ent_embed[:, :H]
    h1 = ent_embed[:, H:]
    r0 = rel_embed[:, :H]
    r1 = rel_embed[:, H:]
    normv = norm

    S0 = _sc_layer(h0, h1, r0, r1, src3, dst3, rel3).reshape(2, NP_, H)
    g0, g1 = _tc_transform(S0, W_r0, normv, split_out=True)
    S1 = _sc_layer(g0, g1, r0, r1, src3, dst3, rel3).reshape(2, NP_, H)
    return _tc_transform(S1, W_r1, normv, split_out=False)
